# baseline (speedup 1.0000x reference)
import functools

import jax
import jax.numpy as jnp
from jax import lax
from jax.experimental import pallas as pl
from jax.experimental.pallas import tpu as pltpu
from jax.experimental.pallas import tpu_sc as plsc

HIDDEN = 768
INTER = 1536
E = 64
B = 4
S = 2048
N = B * S
COEF = 0.001

TB = 256
NT = N // TB
BLK = 256
NB = N // BLK + E
P = NB * BLK


def _route_body(x_ref, wr_ref, br_ref, pos_ref, be_ref, used_ref,
                aux_ref, selbuf_ref, counts_ref, psum_ref, cursor_ref):
    p = pl.program_id(0)
    i = pl.program_id(1)

    @pl.when(p == 0)
    def _():
        logits = jnp.dot(x_ref[...], wr_ref[...],
                         preferred_element_type=jnp.float32)
        logits = logits + br_ref[...][None, :]
        m = jnp.max(logits, axis=1, keepdims=True)
        eids = jax.lax.broadcasted_iota(jnp.int32, logits.shape, 1)
        sel = jnp.min(jnp.where(logits == m, eids, E), axis=1)
        selbuf_ref[i] = sel[None, :]
        ex = jnp.exp(logits - m)
        probs = ex / jnp.sum(ex, axis=1, keepdims=True)
        psum_part = jnp.sum(probs, axis=0)
        onehot = (sel[:, None] == eids[:1, :]).astype(jnp.float32)
        cnt_part = jnp.sum(onehot, axis=0)

        @pl.when(i == 0)
        def _():
            counts_ref[...] = cnt_part
            psum_ref[...] = psum_part

        @pl.when(i > 0)
        def _():
            counts_ref[...] = counts_ref[...] + cnt_part
            psum_ref[...] = psum_ref[...] + psum_part

        @pl.when(i == NT - 1)
        def _():
            f = counts_ref[...] / jnp.float32(N)
            pmean = psum_ref[...] / jnp.float32(N)
            aux_ref[0, 0] = (jnp.float32(E) * jnp.sum(f * pmean)
                             * jnp.float32(COEF))

    @pl.when(p == 1)
    def _():
        @pl.when(i == 0)
        def _():
            counts = counts_ref[...]
            padded = jnp.ceil(counts / BLK) * BLK
            used_ref[0] = (jnp.sum(padded) / BLK).astype(jnp.int32)
            r = jax.lax.broadcasted_iota(jnp.int32, (E, E), 0)
            c = jax.lax.broadcasted_iota(jnp.int32, (E, E), 1)
            tril = (c < r).astype(jnp.float32)
            cum = jnp.dot(tril, padded[:, None],
                          preferred_element_type=jnp.float32)[:, 0]
            cursor_ref[...] = cum
            jrow = jax.lax.broadcasted_iota(jnp.int32, (NB, E), 0) * BLK
            be = jnp.sum((cum[None, :] <= jrow).astype(jnp.int32), axis=1) - 1
            be_ref[...] = be

        sel = selbuf_ref[i][0]
        eids = jax.lax.broadcasted_iota(jnp.int32, (TB, E), 1)
        onehot = (sel[:, None] == eids).astype(jnp.float32)
        r = jax.lax.broadcasted_iota(jnp.int32, (TB, TB), 0)
        c = jax.lax.broadcasted_iota(jnp.int32, (TB, TB), 1)
        tril = (c < r).astype(jnp.float32)
        rank = jnp.dot(tril, onehot, preferred_element_type=jnp.float32)
        cur = cursor_ref[...]
        pos = jnp.sum(onehot * (cur[None, :] + rank), axis=1)
        pos_ref[...] = pos.astype(jnp.int32)
        cursor_ref[...] = cur + jnp.sum(onehot, axis=0)


def _route(x2d, Wr, br):
    return pl.pallas_call(
        _route_body,
        grid=(2, NT),
        in_specs=[
            pl.BlockSpec((TB, HIDDEN), lambda p, i: (jnp.where(p == 0, i, 0), 0)),
            pl.BlockSpec((HIDDEN, E), lambda p, i: (0, 0)),
            pl.BlockSpec((E,), lambda p, i: (0,)),
        ],
        out_specs=[
            pl.BlockSpec((TB,), lambda p, i: (jnp.where(p == 0, 0, i),)),
            pl.BlockSpec((NB,), lambda p, i: (0,)),
            pl.BlockSpec((1,), lambda p, i: (0,), memory_space=pltpu.SMEM),
            pl.BlockSpec((1, 1), lambda p, i: (0, 0),
                         memory_space=pltpu.SMEM),
        ],
        out_shape=[
            jax.ShapeDtypeStruct((N,), jnp.int32),
            jax.ShapeDtypeStruct((NB,), jnp.int32),
            jax.ShapeDtypeStruct((1,), jnp.int32),
            jax.ShapeDtypeStruct((1, 1), jnp.float32),
        ],
        scratch_shapes=[
            pltpu.VMEM((NT, 1, TB), jnp.int32),
            pltpu.VMEM((E,), jnp.float32),
            pltpu.VMEM((E,), jnp.float32),
            pltpu.VMEM((E,), jnp.float32),
        ],
    )(x2d, Wr, br)


def _ffn_body(be_ref, used_ref, xs_ref, wup_ref, bup_ref, wdn_ref, bdn_ref,
              ys_ref):
    i = pl.program_id(0)

    @pl.when(i < used_ref[0])
    def _():
        h = jnp.dot(xs_ref[...], wup_ref[0],
                    preferred_element_type=jnp.float32)
        h = h + bup_ref[0]
        a = 0.5 * h * (1.0 + jax.lax.erf(h * jnp.float32(0.7071067811865476)))
        y = jnp.dot(a, wdn_ref[0], preferred_element_type=jnp.float32)
        ys_ref[...] = y + bdn_ref[0]


def _ffn(block_expert, used, xs, W_up, b_up, W_down, b_down):
    grid_spec = pltpu.PrefetchScalarGridSpec(
        num_scalar_prefetch=2,
        grid=(NB,),
        in_specs=[
            pl.BlockSpec((BLK, HIDDEN),
                         lambda i, be, u: (jnp.minimum(i, u[0] - 1), 0)),
            pl.BlockSpec((1, HIDDEN, INTER),
                         lambda i, be, u: (be[jnp.minimum(i, u[0] - 1)], 0, 0)),
            pl.BlockSpec((1, 1, INTER),
                         lambda i, be, u: (be[jnp.minimum(i, u[0] - 1)], 0, 0)),
            pl.BlockSpec((1, INTER, HIDDEN),
                         lambda i, be, u: (be[jnp.minimum(i, u[0] - 1)], 0, 0)),
            pl.BlockSpec((1, 1, HIDDEN),
                         lambda i, be, u: (be[jnp.minimum(i, u[0] - 1)], 0, 0)),
        ],
        out_specs=pl.BlockSpec(
            (BLK, HIDDEN), lambda i, be, u: (jnp.minimum(i, u[0] - 1), 0)),
    )
    return pl.pallas_call(
        _ffn_body,
        grid_spec=grid_spec,
        out_shape=jax.ShapeDtypeStruct((P, HIDDEN), jnp.float32),
    )(block_expert, used, xs, W_up, b_up.reshape(E, 1, INTER),
      W_down, b_down.reshape(E, 1, HIDDEN))


NC = 2
NS = 16
NW = NC * NS
TPW = N // NW
CH = 128

_SC_MESH = plsc.VectorSubcoreMesh(
    core_axis_name="c", subcore_axis_name="s", num_cores=NC, num_subcores=NS)


def _sc_scatter_body(pos_hbm, x_hbm, xs_hbm, idx_v, rows_v, sem):
    wid = lax.axis_index("s") * NC + lax.axis_index("c")
    for k in range(TPW // CH):
        base = wid * TPW + k * CH
        pltpu.sync_copy(pos_hbm.at[pl.ds(base, CH)], idx_v)
        pltpu.async_copy(x_hbm.at[pl.ds(base, CH), :], rows_v, sem).wait()
        pltpu.sync_copy(rows_v, xs_hbm.at[idx_v])


@functools.partial(
    pl.kernel,
    out_type=jax.ShapeDtypeStruct((P, HIDDEN), jnp.float32),
    mesh=_SC_MESH,
    scratch_types=[
        pltpu.VMEM((CH,), jnp.int32),
        pltpu.VMEM((CH, HIDDEN), jnp.float32),
        pltpu.SemaphoreType.DMA,
    ],
)
def _sc_scatter(pos_hbm, x_hbm, xs_hbm, idx_v, rows_v, sem):
    _sc_scatter_body(pos_hbm, x_hbm, xs_hbm, idx_v, rows_v, sem)


def _sc_gather_body(pos_hbm, ys_hbm, out_hbm, idx_v, rows_v, sem):
    wid = lax.axis_index("s") * NC + lax.axis_index("c")
    for k in range(TPW // CH):
        base = wid * TPW + k * CH
        pltpu.sync_copy(pos_hbm.at[pl.ds(base, CH)], idx_v)
        pltpu.async_copy(ys_hbm.at[idx_v], rows_v, sem).wait()
        pltpu.sync_copy(rows_v, out_hbm.at[pl.ds(base, CH), :])


@functools.partial(
    pl.kernel,
    out_type=jax.ShapeDtypeStruct((N, HIDDEN), jnp.float32),
    mesh=_SC_MESH,
    scratch_types=[
        pltpu.VMEM((CH,), jnp.int32),
        pltpu.VMEM((CH, HIDDEN), jnp.float32),
        pltpu.SemaphoreType.DMA,
    ],
)
def _sc_gather(pos_hbm, ys_hbm, out_hbm, idx_v, rows_v, sem):
    _sc_gather_body(pos_hbm, ys_hbm, out_hbm, idx_v, rows_v, sem)


def kernel(x, Wr, br, W_up, b_up, W_down, b_down):
    x2d = x.reshape(N, HIDDEN)
    pos, block_expert, used, aux = _route(x2d, Wr, br)
    xs = _sc_scatter(pos, x2d)
    ys = _ffn(block_expert, used, xs, W_up, b_up, W_down, b_down)
    out = _sc_gather(pos, ys)
    return out.reshape(B, S, HIDDEN), aux[0, 0]


# speedup vs baseline: 1.0375x; 1.0375x over previous
import functools

import jax
import jax.numpy as jnp
from jax import lax
from jax.experimental import pallas as pl
from jax.experimental.pallas import tpu as pltpu
from jax.experimental.pallas import tpu_sc as plsc

HIDDEN = 768
INTER = 1536
E = 64
B = 4
S = 2048
N = B * S
COEF = 0.001

TB = 1024
NT = N // TB
BLK = 256
NB = N // BLK + E
P = NB * BLK


def _route_body(x_ref, wr_ref, br_ref, pos_ref, be_ref, used_ref,
                aux_ref, selbuf_ref, counts_ref, psum_ref, cursor_ref):
    p = pl.program_id(0)
    i = pl.program_id(1)

    @pl.when(p == 0)
    def _():
        logits = jnp.dot(x_ref[...], wr_ref[...],
                         preferred_element_type=jnp.float32)
        logits = logits + br_ref[...][None, :]
        m = jnp.max(logits, axis=1, keepdims=True)
        eids = jax.lax.broadcasted_iota(jnp.int32, logits.shape, 1)
        sel = jnp.min(jnp.where(logits == m, eids, E), axis=1)
        selbuf_ref[i] = sel[None, :]
        ex = jnp.exp(logits - m)
        probs = ex / jnp.sum(ex, axis=1, keepdims=True)
        psum_part = jnp.sum(probs, axis=0)
        onehot = (sel[:, None] == eids[:1, :]).astype(jnp.float32)
        cnt_part = jnp.sum(onehot, axis=0)

        @pl.when(i == 0)
        def _():
            counts_ref[...] = cnt_part
            psum_ref[...] = psum_part

        @pl.when(i > 0)
        def _():
            counts_ref[...] = counts_ref[...] + cnt_part
            psum_ref[...] = psum_ref[...] + psum_part

        @pl.when(i == NT - 1)
        def _():
            f = counts_ref[...] / jnp.float32(N)
            pmean = psum_ref[...] / jnp.float32(N)
            aux_ref[0, 0] = (jnp.float32(E) * jnp.sum(f * pmean)
                             * jnp.float32(COEF))

    @pl.when(p == 1)
    def _():
        @pl.when(i == 0)
        def _():
            counts = counts_ref[...]
            padded = jnp.ceil(counts / BLK) * BLK
            used_ref[0] = (jnp.sum(padded) / BLK).astype(jnp.int32)
            r = jax.lax.broadcasted_iota(jnp.int32, (E, E), 0)
            c = jax.lax.broadcasted_iota(jnp.int32, (E, E), 1)
            tril = (c < r).astype(jnp.float32)
            cum = jnp.dot(tril, padded[:, None],
                          preferred_element_type=jnp.float32)[:, 0]
            cursor_ref[...] = cum
            jrow = jax.lax.broadcasted_iota(jnp.int32, (NB, E), 0) * BLK
            be = jnp.sum((cum[None, :] <= jrow).astype(jnp.int32), axis=1) - 1
            be_ref[...] = be

        sel = selbuf_ref[i][0]
        eids = jax.lax.broadcasted_iota(jnp.int32, (TB, E), 1)
        onehot = (sel[:, None] == eids).astype(jnp.float32)
        r = jax.lax.broadcasted_iota(jnp.int32, (TB, TB), 0)
        c = jax.lax.broadcasted_iota(jnp.int32, (TB, TB), 1)
        tril = (c < r).astype(jnp.float32)
        rank = jnp.dot(tril, onehot, preferred_element_type=jnp.float32)
        cur = cursor_ref[...]
        pos = jnp.sum(onehot * (cur[None, :] + rank), axis=1)
        pos_ref[...] = pos.astype(jnp.int32)
        cursor_ref[...] = cur + jnp.sum(onehot, axis=0)


def _route(x2d, Wr, br):
    return pl.pallas_call(
        _route_body,
        grid=(2, NT),
        in_specs=[
            pl.BlockSpec((TB, HIDDEN), lambda p, i: (jnp.where(p == 0, i, 0), 0)),
            pl.BlockSpec((HIDDEN, E), lambda p, i: (0, 0)),
            pl.BlockSpec((E,), lambda p, i: (0,)),
        ],
        out_specs=[
            pl.BlockSpec((TB,), lambda p, i: (jnp.where(p == 0, 0, i),)),
            pl.BlockSpec((NB,), lambda p, i: (0,)),
            pl.BlockSpec((1,), lambda p, i: (0,), memory_space=pltpu.SMEM),
            pl.BlockSpec((1, 1), lambda p, i: (0, 0),
                         memory_space=pltpu.SMEM),
        ],
        out_shape=[
            jax.ShapeDtypeStruct((N,), jnp.int32),
            jax.ShapeDtypeStruct((NB,), jnp.int32),
            jax.ShapeDtypeStruct((1,), jnp.int32),
            jax.ShapeDtypeStruct((1, 1), jnp.float32),
        ],
        scratch_shapes=[
            pltpu.VMEM((NT, 1, TB), jnp.int32),
            pltpu.VMEM((E,), jnp.float32),
            pltpu.VMEM((E,), jnp.float32),
            pltpu.VMEM((E,), jnp.float32),
        ],
    )(x2d, Wr, br)


def _ffn_body(be_ref, used_ref, xs_ref, wup_ref, bup_ref, wdn_ref, bdn_ref,
              ys_ref):
    i = pl.program_id(0)

    @pl.when(i < used_ref[0])
    def _():
        h = jnp.dot(xs_ref[...], wup_ref[0],
                    preferred_element_type=jnp.float32)
        h = h + bup_ref[0]
        a = 0.5 * h * (1.0 + jax.lax.erf(h * jnp.float32(0.7071067811865476)))
        y = jnp.dot(a, wdn_ref[0], preferred_element_type=jnp.float32)
        ys_ref[...] = y + bdn_ref[0]


def _ffn(block_expert, used, xs, W_up, b_up, W_down, b_down):
    grid_spec = pltpu.PrefetchScalarGridSpec(
        num_scalar_prefetch=2,
        grid=(NB,),
        in_specs=[
            pl.BlockSpec((BLK, HIDDEN),
                         lambda i, be, u: (jnp.minimum(i, u[0] - 1), 0)),
            pl.BlockSpec((1, HIDDEN, INTER),
                         lambda i, be, u: (be[jnp.minimum(i, u[0] - 1)], 0, 0)),
            pl.BlockSpec((1, 1, INTER),
                         lambda i, be, u: (be[jnp.minimum(i, u[0] - 1)], 0, 0)),
            pl.BlockSpec((1, INTER, HIDDEN),
                         lambda i, be, u: (be[jnp.minimum(i, u[0] - 1)], 0, 0)),
            pl.BlockSpec((1, 1, HIDDEN),
                         lambda i, be, u: (be[jnp.minimum(i, u[0] - 1)], 0, 0)),
        ],
        out_specs=pl.BlockSpec(
            (BLK, HIDDEN), lambda i, be, u: (jnp.minimum(i, u[0] - 1), 0)),
    )
    return pl.pallas_call(
        _ffn_body,
        grid_spec=grid_spec,
        out_shape=jax.ShapeDtypeStruct((P, HIDDEN), jnp.float32),
    )(block_expert, used, xs, W_up, b_up.reshape(E, 1, INTER),
      W_down, b_down.reshape(E, 1, HIDDEN))


NC = 2
NS = 16
NW = NC * NS
TPW = N // NW
CH = 128

_SC_MESH = plsc.VectorSubcoreMesh(
    core_axis_name="c", subcore_axis_name="s", num_cores=NC, num_subcores=NS)


def _sc_scatter_body(pos_hbm, x_hbm, xs_hbm, idx_v, rows_v, sem):
    wid = lax.axis_index("s") * NC + lax.axis_index("c")
    for k in range(TPW // CH):
        base = wid * TPW + k * CH
        pltpu.sync_copy(pos_hbm.at[pl.ds(base, CH)], idx_v)
        pltpu.async_copy(x_hbm.at[pl.ds(base, CH), :], rows_v, sem).wait()
        pltpu.sync_copy(rows_v, xs_hbm.at[idx_v])


@functools.partial(
    pl.kernel,
    out_type=jax.ShapeDtypeStruct((P, HIDDEN), jnp.float32),
    mesh=_SC_MESH,
    scratch_types=[
        pltpu.VMEM((CH,), jnp.int32),
        pltpu.VMEM((CH, HIDDEN), jnp.float32),
        pltpu.SemaphoreType.DMA,
    ],
)
def _sc_scatter(pos_hbm, x_hbm, xs_hbm, idx_v, rows_v, sem):
    _sc_scatter_body(pos_hbm, x_hbm, xs_hbm, idx_v, rows_v, sem)


def _sc_gather_body(pos_hbm, ys_hbm, out_hbm, idx_v, rows_v, sem):
    wid = lax.axis_index("s") * NC + lax.axis_index("c")
    for k in range(TPW // CH):
        base = wid * TPW + k * CH
        pltpu.sync_copy(pos_hbm.at[pl.ds(base, CH)], idx_v)
        pltpu.async_copy(ys_hbm.at[idx_v], rows_v, sem).wait()
        pltpu.sync_copy(rows_v, out_hbm.at[pl.ds(base, CH), :])


@functools.partial(
    pl.kernel,
    out_type=jax.ShapeDtypeStruct((N, HIDDEN), jnp.float32),
    mesh=_SC_MESH,
    scratch_types=[
        pltpu.VMEM((CH,), jnp.int32),
        pltpu.VMEM((CH, HIDDEN), jnp.float32),
        pltpu.SemaphoreType.DMA,
    ],
)
def _sc_gather(pos_hbm, ys_hbm, out_hbm, idx_v, rows_v, sem):
    _sc_gather_body(pos_hbm, ys_hbm, out_hbm, idx_v, rows_v, sem)


def kernel(x, Wr, br, W_up, b_up, W_down, b_down):
    x2d = x.reshape(N, HIDDEN)
    pos, block_expert, used, aux = _route(x2d, Wr, br)
    xs = _sc_scatter(pos, x2d)
    ys = _ffn(block_expert, used, xs, W_up, b_up, W_down, b_down)
    out = _sc_gather(pos, ys)
    return out.reshape(B, S, HIDDEN), aux[0, 0]


# speedup vs baseline: 1.0435x; 1.0058x over previous
import functools

import jax
import jax.numpy as jnp
from jax import lax
from jax.experimental import pallas as pl
from jax.experimental.pallas import tpu as pltpu
from jax.experimental.pallas import tpu_sc as plsc

HIDDEN = 768
INTER = 1536
E = 64
B = 4
S = 2048
N = B * S
COEF = 0.001

TB = 512
NT = N // TB
BLK = 256
NB = N // BLK + E
P = NB * BLK


def _route_body(x_ref, wr_ref, br_ref, pos_ref, be_ref, used_ref,
                aux_ref, selbuf_ref, counts_ref, psum_ref, cursor_ref):
    p = pl.program_id(0)
    i = pl.program_id(1)

    @pl.when(p == 0)
    def _():
        logits = jnp.dot(x_ref[...], wr_ref[...],
                         preferred_element_type=jnp.float32)
        logits = logits + br_ref[...][None, :]
        m = jnp.max(logits, axis=1, keepdims=True)
        eids = jax.lax.broadcasted_iota(jnp.int32, logits.shape, 1)
        sel = jnp.min(jnp.where(logits == m, eids, E), axis=1)
        selbuf_ref[i] = sel[None, :]
        ex = jnp.exp(logits - m)
        probs = ex / jnp.sum(ex, axis=1, keepdims=True)
        psum_part = jnp.sum(probs, axis=0)
        onehot = (sel[:, None] == eids[:1, :]).astype(jnp.float32)
        cnt_part = jnp.sum(onehot, axis=0)

        @pl.when(i == 0)
        def _():
            counts_ref[...] = cnt_part
            psum_ref[...] = psum_part

        @pl.when(i > 0)
        def _():
            counts_ref[...] = counts_ref[...] + cnt_part
            psum_ref[...] = psum_ref[...] + psum_part

        @pl.when(i == NT - 1)
        def _():
            f = counts_ref[...] / jnp.float32(N)
            pmean = psum_ref[...] / jnp.float32(N)
            aux_ref[0, 0] = (jnp.float32(E) * jnp.sum(f * pmean)
                             * jnp.float32(COEF))

    @pl.when(p == 1)
    def _():
        @pl.when(i == 0)
        def _():
            counts = counts_ref[...]
            padded = jnp.ceil(counts / BLK) * BLK
            used_ref[0] = (jnp.sum(padded) / BLK).astype(jnp.int32)
            r = jax.lax.broadcasted_iota(jnp.int32, (E, E), 0)
            c = jax.lax.broadcasted_iota(jnp.int32, (E, E), 1)
            tril = (c < r).astype(jnp.float32)
            cum = jnp.dot(tril, padded[:, None],
                          preferred_element_type=jnp.float32)[:, 0]
            cursor_ref[...] = cum
            jrow = jax.lax.broadcasted_iota(jnp.int32, (NB, E), 0) * BLK
            be = jnp.sum((cum[None, :] <= jrow).astype(jnp.int32), axis=1) - 1
            be_ref[...] = be

        sel = selbuf_ref[i][0]
        eids = jax.lax.broadcasted_iota(jnp.int32, (TB, E), 1)
        onehot = (sel[:, None] == eids).astype(jnp.float32)
        r = jax.lax.broadcasted_iota(jnp.int32, (TB, TB), 0)
        c = jax.lax.broadcasted_iota(jnp.int32, (TB, TB), 1)
        tril = (c < r).astype(jnp.float32)
        rank = jnp.dot(tril, onehot, preferred_element_type=jnp.float32)
        cur = cursor_ref[...]
        pos = jnp.sum(onehot * (cur[None, :] + rank), axis=1)
        pos_ref[...] = pos.astype(jnp.int32)
        cursor_ref[...] = cur + jnp.sum(onehot, axis=0)


def _route(x2d, Wr, br):
    return pl.pallas_call(
        _route_body,
        grid=(2, NT),
        in_specs=[
            pl.BlockSpec((TB, HIDDEN), lambda p, i: (jnp.where(p == 0, i, 0), 0)),
            pl.BlockSpec((HIDDEN, E), lambda p, i: (0, 0)),
            pl.BlockSpec((E,), lambda p, i: (0,)),
        ],
        out_specs=[
            pl.BlockSpec((TB,), lambda p, i: (jnp.where(p == 0, 0, i),)),
            pl.BlockSpec((NB,), lambda p, i: (0,)),
            pl.BlockSpec((1,), lambda p, i: (0,), memory_space=pltpu.SMEM),
            pl.BlockSpec((1, 1), lambda p, i: (0, 0),
                         memory_space=pltpu.SMEM),
        ],
        out_shape=[
            jax.ShapeDtypeStruct((N,), jnp.int32),
            jax.ShapeDtypeStruct((NB,), jnp.int32),
            jax.ShapeDtypeStruct((1,), jnp.int32),
            jax.ShapeDtypeStruct((1, 1), jnp.float32),
        ],
        scratch_shapes=[
            pltpu.VMEM((NT, 1, TB), jnp.int32),
            pltpu.VMEM((E,), jnp.float32),
            pltpu.VMEM((E,), jnp.float32),
            pltpu.VMEM((E,), jnp.float32),
        ],
    )(x2d, Wr, br)


def _ffn_body(be_ref, used_ref, xs_ref, wup_ref, bup_ref, wdn_ref, bdn_ref,
              ys_ref):
    i = pl.program_id(0)

    @pl.when(i < used_ref[0])
    def _():
        h = jnp.dot(xs_ref[...], wup_ref[0],
                    preferred_element_type=jnp.float32)
        h = h + bup_ref[0]
        a = 0.5 * h * (1.0 + jax.lax.erf(h * jnp.float32(0.7071067811865476)))
        y = jnp.dot(a, wdn_ref[0], preferred_element_type=jnp.float32)
        ys_ref[...] = y + bdn_ref[0]


def _ffn(block_expert, used, xs, W_up, b_up, W_down, b_down):
    grid_spec = pltpu.PrefetchScalarGridSpec(
        num_scalar_prefetch=2,
        grid=(NB,),
        in_specs=[
            pl.BlockSpec((BLK, HIDDEN),
                         lambda i, be, u: (jnp.minimum(i, u[0] - 1), 0)),
            pl.BlockSpec((1, HIDDEN, INTER),
                         lambda i, be, u: (be[jnp.minimum(i, u[0] - 1)], 0, 0)),
            pl.BlockSpec((1, 1, INTER),
                         lambda i, be, u: (be[jnp.minimum(i, u[0] - 1)], 0, 0)),
            pl.BlockSpec((1, INTER, HIDDEN),
                         lambda i, be, u: (be[jnp.minimum(i, u[0] - 1)], 0, 0)),
            pl.BlockSpec((1, 1, HIDDEN),
                         lambda i, be, u: (be[jnp.minimum(i, u[0] - 1)], 0, 0)),
        ],
        out_specs=pl.BlockSpec(
            (BLK, HIDDEN), lambda i, be, u: (jnp.minimum(i, u[0] - 1), 0)),
    )
    return pl.pallas_call(
        _ffn_body,
        grid_spec=grid_spec,
        out_shape=jax.ShapeDtypeStruct((P, HIDDEN), jnp.float32),
    )(block_expert, used, xs, W_up, b_up.reshape(E, 1, INTER),
      W_down, b_down.reshape(E, 1, HIDDEN))


NC = 2
NS = 16
NW = NC * NS
TPW = N // NW
CH = 64
NCH = TPW // CH

_SC_MESH = plsc.VectorSubcoreMesh(
    core_axis_name="c", subcore_axis_name="s", num_cores=NC, num_subcores=NS)

_SC_SCRATCH = (
    [pltpu.VMEM((CH,), jnp.int32) for _ in range(NCH)]
    + [pltpu.VMEM((CH, HIDDEN), jnp.float32) for _ in range(2)]
    + [pltpu.SemaphoreType.DMA((NCH,)), pltpu.SemaphoreType.DMA((2,))]
)


@functools.partial(
    pl.kernel,
    out_type=jax.ShapeDtypeStruct((P, HIDDEN), jnp.float32),
    mesh=_SC_MESH,
    scratch_types=_SC_SCRATCH,
)
def _sc_scatter(pos_hbm, x_hbm, xs_hbm, i0, i1, i2, i3, r0, r1, isem, rsem):
    wid = lax.axis_index("s") * NC + lax.axis_index("c")
    base = wid * TPW
    idx = [i0, i1, i2, i3]
    rows = [r0, r1]
    for j in range(NCH):
        pltpu.make_async_copy(pos_hbm.at[pl.ds(base + j * CH, CH)], idx[j],
                         isem.at[j]).start()
    for s in range(2):
        pltpu.make_async_copy(x_hbm.at[pl.ds(base + s * CH, CH), :], rows[s],
                         rsem.at[s]).start()
    for j in range(NCH):
        s = j % 2
        pltpu.make_async_copy(x_hbm.at[pl.ds(base + j * CH, CH), :], rows[s],
                         rsem.at[s]).wait()
        pltpu.make_async_copy(pos_hbm.at[pl.ds(base + j * CH, CH)], idx[j],
                         isem.at[j]).wait()
        pltpu.sync_copy(rows[s], xs_hbm.at[idx[j]])
        if j + 2 < NCH:
            pltpu.make_async_copy(x_hbm.at[pl.ds(base + (j + 2) * CH, CH), :],
                             rows[s], rsem.at[s]).start()


@functools.partial(
    pl.kernel,
    out_type=jax.ShapeDtypeStruct((N, HIDDEN), jnp.float32),
    mesh=_SC_MESH,
    scratch_types=_SC_SCRATCH,
)
def _sc_gather(pos_hbm, ys_hbm, out_hbm, i0, i1, i2, i3, r0, r1, isem, rsem):
    wid = lax.axis_index("s") * NC + lax.axis_index("c")
    base = wid * TPW
    idx = [i0, i1, i2, i3]
    rows = [r0, r1]
    for j in range(NCH):
        pltpu.make_async_copy(pos_hbm.at[pl.ds(base + j * CH, CH)], idx[j],
                         isem.at[j]).start()
    for s in range(2):
        pltpu.make_async_copy(pos_hbm.at[pl.ds(base + s * CH, CH)], idx[s],
                         isem.at[s]).wait()
        pltpu.make_async_copy(ys_hbm.at[idx[s]], rows[s], rsem.at[s]).start()
    for j in range(NCH):
        s = j % 2
        pltpu.make_async_copy(ys_hbm.at[idx[j]], rows[s], rsem.at[s]).wait()
        pltpu.sync_copy(rows[s], out_hbm.at[pl.ds(base + j * CH, CH), :])
        if j + 2 < NCH:
            pltpu.make_async_copy(pos_hbm.at[pl.ds(base + (j + 2) * CH, CH)],
                             idx[j + 2], isem.at[j + 2]).wait()
            pltpu.make_async_copy(ys_hbm.at[idx[j + 2]], rows[s],
                             rsem.at[s]).start()


def kernel(x, Wr, br, W_up, b_up, W_down, b_down):
    x2d = x.reshape(N, HIDDEN)
    pos, block_expert, used, aux = _route(x2d, Wr, br)
    xs = _sc_scatter(pos, x2d)
    ys = _ffn(block_expert, used, xs, W_up, b_up, W_down, b_down)
    out = _sc_gather(pos, ys)
    return out.reshape(B, S, HIDDEN), aux[0, 0]
